# baseline (speedup 1.0000x reference)
import jax
import jax.numpy as jnp
from jax.experimental import pallas as pl
import jax.experimental.pallas.tpu as pltpu

N = 4096
NFEAT = 128
NHID = 32
NCLASS = 16
TILE = 256

_DEFAULT = jax.lax.Precision.DEFAULT


def _dot(a, b):
    return jax.lax.dot_general(a, b, (((1,), (0,)), ((), ())),
                               precision=_DEFAULT,
                               preferred_element_type=jnp.float32)


def _gcn_kernel(x_ref, adj_ref, w1_ref, b1_ref, w2_ref, b2_ref, out_ref,
                s1_ref, h_ref, s2_ref):
    p = pl.program_id(0)
    t = pl.program_id(1)

    @pl.when((p == 0) & (t == 0))
    def _():
        s1_ref[...] = _dot(x_ref[...], w1_ref[...])

    @pl.when(p == 0)
    def _():
        h = _dot(adj_ref[0], s1_ref[...])
        h_ref[pl.ds(t * TILE, TILE), :] = jnp.maximum(h + b1_ref[...], 0.0)

    @pl.when((p == 1) & (t == 0))
    def _():
        s2_ref[...] = _dot(h_ref[...], w2_ref[...])

    @pl.when(p == 1)
    def _():
        o = _dot(adj_ref[0], s2_ref[...]) + b2_ref[...]
        out_ref[...] = jnp.minimum(o, 0.0) - jnp.log1p(jnp.exp(-jnp.abs(o)))


@jax.jit
def kernel(x, adj_list, W1, b1, W2, b2):
    grid = (2, N // TILE)
    return pl.pallas_call(
        _gcn_kernel,
        grid=grid,
        in_specs=[
            pl.BlockSpec((N, NFEAT), lambda p, t: (0, 0)),
            pl.BlockSpec((1, TILE, N), lambda p, t: (p, t, 0)),
            pl.BlockSpec((NFEAT, NHID), lambda p, t: (0, 0)),
            pl.BlockSpec((1, NHID), lambda p, t: (0, 0)),
            pl.BlockSpec((NHID, NCLASS), lambda p, t: (0, 0)),
            pl.BlockSpec((1, NCLASS), lambda p, t: (0, 0)),
        ],
        out_specs=pl.BlockSpec((TILE, NCLASS), lambda p, t: (p * t, 0)),
        out_shape=jax.ShapeDtypeStruct((N, NCLASS), jnp.float32),
        scratch_shapes=[
            pltpu.VMEM((N, NHID), jnp.float32),
            pltpu.VMEM((N, NHID), jnp.float32),
            pltpu.VMEM((N, NCLASS), jnp.float32),
        ],
    )(x, adj_list, W1, b1.reshape(1, NHID), W2, b2.reshape(1, NCLASS))


# speedup vs baseline: 1.1751x; 1.1751x over previous
import jax
import jax.numpy as jnp
from jax.experimental import pallas as pl
import jax.experimental.pallas.tpu as pltpu

N = 4096
NFEAT = 128
NHID = 32
NCLASS = 16
TILE = 512

_DEFAULT = jax.lax.Precision.DEFAULT


def _dot(a, b):
    return jax.lax.dot_general(a, b, (((1,), (0,)), ((), ())),
                               precision=_DEFAULT,
                               preferred_element_type=jnp.float32)


def _gcn_kernel(x_ref, adj_ref, w1_ref, b1_ref, w2_ref, b2_ref, out_ref,
                s1_ref, h_ref, s2_ref):
    p = pl.program_id(0)
    t = pl.program_id(1)

    @pl.when((p == 0) & (t == 0))
    def _():
        s1_ref[...] = _dot(x_ref[...], w1_ref[...])

    @pl.when(p == 0)
    def _():
        h = _dot(adj_ref[0], s1_ref[...])
        h_ref[pl.ds(t * TILE, TILE), :] = jnp.maximum(h + b1_ref[...], 0.0)

    @pl.when((p == 1) & (t == 0))
    def _():
        s2_ref[...] = _dot(h_ref[...], w2_ref[...])

    @pl.when(p == 1)
    def _():
        o = _dot(adj_ref[0], s2_ref[...]) + b2_ref[...]
        out_ref[...] = jnp.minimum(o, 0.0) - jnp.log1p(jnp.exp(-jnp.abs(o)))


@jax.jit
def kernel(x, adj_list, W1, b1, W2, b2):
    grid = (2, N // TILE)
    return pl.pallas_call(
        _gcn_kernel,
        grid=grid,
        in_specs=[
            pl.BlockSpec((N, NFEAT), lambda p, t: (0, 0)),
            pl.BlockSpec((1, TILE, N), lambda p, t: (p, t, 0)),
            pl.BlockSpec((NFEAT, NHID), lambda p, t: (0, 0)),
            pl.BlockSpec((1, NHID), lambda p, t: (0, 0)),
            pl.BlockSpec((NHID, NCLASS), lambda p, t: (0, 0)),
            pl.BlockSpec((1, NCLASS), lambda p, t: (0, 0)),
        ],
        out_specs=pl.BlockSpec((TILE, NCLASS), lambda p, t: (p * t, 0)),
        out_shape=jax.ShapeDtypeStruct((N, NCLASS), jnp.float32),
        scratch_shapes=[
            pltpu.VMEM((N, NHID), jnp.float32),
            pltpu.VMEM((N, NHID), jnp.float32),
            pltpu.VMEM((N, NCLASS), jnp.float32),
        ],
    )(x, adj_list, W1, b1.reshape(1, NHID), W2, b2.reshape(1, NCLASS))
